# Initial kernel scaffold; baseline (speedup 1.0000x reference)
import jax
import jax.numpy as jnp
from jax.experimental import pallas as pl


def kernel(inp, vx, gx, vy, gy):
    raise NotImplementedError("write your pallas kernel here")


# speedup vs baseline: 2.3346x; 2.3346x over previous
import jax
import jax.numpy as jnp
from jax.experimental import pallas as pl


def _body(vx_ref, gx_ref, vy_ref, gy_ref, out_ref):
    H = vx_ref.shape[0]
    W = vy_ref.shape[0]
    vx = vx_ref[...]
    wx = gx_ref[...] * vx * jax.lax.rsqrt(jnp.sum(vx * vx, axis=1, keepdims=True))
    vy = vy_ref[...]
    wy = gy_ref[...] * vy * jax.lax.rsqrt(jnp.sum(vy * vy, axis=1, keepdims=True))
    xblock = jnp.tile(wx, (W, 1))
    yblock = jnp.repeat(wy, H, axis=0)
    out_ref[0] = jnp.concatenate([xblock, yblock], axis=1)


def kernel(inp, vx, gx, vy, gy):
    b = inp.shape[0]
    H, D = vx.shape
    W = vy.shape[0]
    full = lambda s: pl.BlockSpec(s, lambda i: (0,) * len(s))
    return pl.pallas_call(
        _body,
        grid=(b,),
        in_specs=[full((H, D)), full((H, 1)), full((W, D)), full((W, 1))],
        out_specs=pl.BlockSpec((1, W * H, 2 * D), lambda i: (i, 0, 0)),
        out_shape=jax.ShapeDtypeStruct((b, W * H, 2 * D), jnp.float32),
    )(vx, gx, vy, gy)
